# baseline (speedup 1.0000x reference)
import jax
import jax.numpy as jnp
from jax import lax
from jax.experimental import pallas as pl
from jax.experimental.pallas import tpu as pltpu

_BT = 512
_E = 8


def _dot(a, b):
    return lax.dot_general(a, b, (((1,), (0,)), ((), ())),
                           preferred_element_type=jnp.float32)


def _moe_kernel(x_ref, w13_hbm, w2_hbm, o_ref,
                w13b, w2b, st13, st2, wst13, wst2,
                sem13, sem2, wsem13, wsem2):
    e = pl.program_id(0)
    t = pl.program_id(1)
    nt = pl.num_programs(1)
    g = e * nt + t
    h13 = w13_hbm.shape[1] // nt
    h2 = w2_hbm.shape[1] // nt

    def cp13(gp):
        src = jnp.minimum(gp // nt + 1, _E - 1)
        return pltpu.make_async_copy(
            w13_hbm.at[src, pl.ds((gp % nt) * h13, h13), :],
            st13.at[gp % 2], sem13.at[gp % 2])

    def cp2(gp):
        src = jnp.minimum(gp // nt + 1, _E - 1)
        return pltpu.make_async_copy(
            w2_hbm.at[src, pl.ds((gp % nt) * h2, h2), :],
            st2.at[gp % 2], sem2.at[gp % 2])

    @pl.when(g == 0)
    def _warmup():
        for k in range(_E):
            pltpu.make_async_copy(
                w13_hbm.at[0, pl.ds(k * h13, h13), :], wst13, wsem13).start()
            pltpu.make_async_copy(
                w2_hbm.at[0, pl.ds(k * h2, h2), :], wst2, wsem2).start()
            pltpu.make_async_copy(
                w13_hbm.at[0, pl.ds(k * h13, h13), :], wst13, wsem13).wait()
            pltpu.make_async_copy(
                w2_hbm.at[0, pl.ds(k * h2, h2), :], wst2, wsem2).wait()
            w13b[0, pl.ds(k * h13, h13), :] = wst13[...].astype(jnp.bfloat16)
            w2b[0, pl.ds(k * h2, h2), :] = wst2[...].astype(jnp.bfloat16)
        cp13(0).start()
        cp2(0).start()

    last_real = (_E - 1) * nt - 1

    @pl.when(g + 1 <= last_real)
    def _issue():
        cp13(g + 1).start()
        cp2(g + 1).start()

    x = x_ref[0]
    i = w2b.shape[1]
    cw = i // 8
    acc = None
    for c in range(8):
        gc = _dot(x, w13b[e % 2, :, c * cw:(c + 1) * cw])
        uc = _dot(x, w13b[e % 2, :, i + c * cw:i + (c + 1) * cw])
        actc = gc * jax.nn.sigmoid(gc) * uc
        pc = _dot(actc, w2b[e % 2, c * cw:(c + 1) * cw, :])
        acc = pc if acc is None else acc + pc
    o_ref[0] = acc

    @pl.when(g <= last_real)
    def _wait_cast():
        cp13(g).wait()
        cp2(g).wait()
        b = (g // nt + 1) % 2
        k13 = (g % nt) * h13
        k2 = (g % nt) * h2
        w13b[b, pl.ds(k13, h13), :] = st13[g % 2].astype(jnp.bfloat16)
        w2b[b, pl.ds(k2, h2), :] = st2[g % 2].astype(jnp.bfloat16)


def kernel(x, tokens_per_expert, decoding, W13, W2):
    T, H = x.shape
    E, _, I2 = W13.shape
    I = I2 // 2
    S = T // E
    nt = S // _BT

    xb = x.reshape(E, S, H)

    out = pl.pallas_call(
        _moe_kernel,
        grid=(E, nt),
        in_specs=[
            pl.BlockSpec((1, _BT, H), lambda e, t: (e, t, 0)),
            pl.BlockSpec(memory_space=pltpu.MemorySpace.HBM),
            pl.BlockSpec(memory_space=pltpu.MemorySpace.HBM),
        ],
        out_specs=pl.BlockSpec((1, _BT, H), lambda e, t: (e, t, 0)),
        out_shape=jax.ShapeDtypeStruct((E, S, H), jnp.float32),
        scratch_shapes=[
            pltpu.VMEM((2, H, I2), jnp.bfloat16),
            pltpu.VMEM((2, I, H), jnp.bfloat16),
            pltpu.VMEM((2, H // nt, I2), jnp.float32),
            pltpu.VMEM((2, I // nt, H), jnp.float32),
            pltpu.VMEM((H // nt, I2), jnp.float32),
            pltpu.VMEM((I // nt, H), jnp.float32),
            pltpu.SemaphoreType.DMA((2,)),
            pltpu.SemaphoreType.DMA((2,)),
            pltpu.SemaphoreType.DMA,
            pltpu.SemaphoreType.DMA,
        ],
        compiler_params=pltpu.CompilerParams(
            vmem_limit_bytes=128 * 1024 * 1024,
        ),
    )(xb, W13, W2)
    return out.reshape(T, H)


# speedup vs baseline: 1.8907x; 1.8907x over previous
import jax
import jax.numpy as jnp
from jax import lax
from jax.experimental import pallas as pl
from jax.experimental.pallas import tpu as pltpu

_BT = 512
_E = 8


def _dot(a, b):
    return lax.dot_general(a, b, (((1,), (0,)), ((), ())),
                           preferred_element_type=jnp.float32)


def _moe_kernel(x_ref, w13_hbm, w2_hbm, o_ref,
                w13b, w2b, st13, st2, wst13, wst2,
                sem13, sem2, wsem13, wsem2):
    e = pl.program_id(0)
    t = pl.program_id(1)
    nt = pl.num_programs(1)
    g = e * nt + t
    h13 = w13_hbm.shape[1] // nt
    h2 = w2_hbm.shape[1] // nt

    def cp13(gp):
        src = jnp.minimum(gp // nt + 1, _E - 1)
        return pltpu.make_async_copy(
            w13_hbm.at[src, pl.ds((gp % nt) * h13, h13), :],
            st13.at[gp % 2], sem13.at[gp % 2])

    def cp2(gp):
        src = jnp.minimum(gp // nt + 1, _E - 1)
        return pltpu.make_async_copy(
            w2_hbm.at[src, pl.ds((gp % nt) * h2, h2), :],
            st2.at[gp % 2], sem2.at[gp % 2])

    @pl.when(g == 0)
    def _warmup():
        for k in range(_E):
            pltpu.make_async_copy(
                w13_hbm.at[0, pl.ds(k * h13, h13), :], wst13, wsem13).start()
            pltpu.make_async_copy(
                w2_hbm.at[0, pl.ds(k * h2, h2), :], wst2, wsem2).start()
            pltpu.make_async_copy(
                w13_hbm.at[0, pl.ds(k * h13, h13), :], wst13, wsem13).wait()
            pltpu.make_async_copy(
                w2_hbm.at[0, pl.ds(k * h2, h2), :], wst2, wsem2).wait()
            w13b[0, pl.ds(k * h13, h13), :] = wst13[...].astype(jnp.bfloat16)
            w2b[0, pl.ds(k * h2, h2), :] = wst2[...].astype(jnp.bfloat16)
        cp13(0).start()
        cp2(0).start()

    last_real = (_E - 1) * nt - 1

    @pl.when(g + 1 <= last_real)
    def _issue():
        cp13(g + 1).start()
        cp2(g + 1).start()

    x = x_ref[0]
    i = w2b.shape[1]
    cw = i // 4
    acc = None
    for c in range(4):
        gc = _dot(x, w13b[e % 2, :, c * cw:(c + 1) * cw])
        uc = _dot(x, w13b[e % 2, :, i + c * cw:i + (c + 1) * cw])
        actc = gc * jax.nn.sigmoid(gc) * uc
        pc = _dot(actc, w2b[e % 2, c * cw:(c + 1) * cw, :])
        acc = pc if acc is None else acc + pc
    o_ref[0] = acc

    @pl.when(g <= last_real)
    def _wait_cast():
        cp13(g).wait()
        cp2(g).wait()
        b = (g // nt + 1) % 2
        k13 = (g % nt) * h13
        k2 = (g % nt) * h2
        w13b[b, pl.ds(k13, h13), :] = st13[g % 2].astype(jnp.bfloat16)
        w2b[b, pl.ds(k2, h2), :] = st2[g % 2].astype(jnp.bfloat16)


def kernel(x, tokens_per_expert, decoding, W13, W2):
    T, H = x.shape
    E, _, I2 = W13.shape
    I = I2 // 2
    S = T // E
    nt = S // _BT

    xb = x.reshape(E, S, H)

    out = pl.pallas_call(
        _moe_kernel,
        grid=(E, nt),
        in_specs=[
            pl.BlockSpec((1, _BT, H), lambda e, t: (e, t, 0)),
            pl.BlockSpec(memory_space=pltpu.MemorySpace.HBM),
            pl.BlockSpec(memory_space=pltpu.MemorySpace.HBM),
        ],
        out_specs=pl.BlockSpec((1, _BT, H), lambda e, t: (e, t, 0)),
        out_shape=jax.ShapeDtypeStruct((E, S, H), jnp.float32),
        scratch_shapes=[
            pltpu.VMEM((2, H, I2), jnp.bfloat16),
            pltpu.VMEM((2, I, H), jnp.bfloat16),
            pltpu.VMEM((2, H // nt, I2), jnp.float32),
            pltpu.VMEM((2, I // nt, H), jnp.float32),
            pltpu.VMEM((H // nt, I2), jnp.float32),
            pltpu.VMEM((I // nt, H), jnp.float32),
            pltpu.SemaphoreType.DMA((2,)),
            pltpu.SemaphoreType.DMA((2,)),
            pltpu.SemaphoreType.DMA,
            pltpu.SemaphoreType.DMA,
        ],
        compiler_params=pltpu.CompilerParams(
            vmem_limit_bytes=128 * 1024 * 1024,
        ),
    )(xb, W13, W2)
    return out.reshape(T, H)
